# baseline (speedup 1.0000x reference)
import functools

import jax
import jax.numpy as jnp
from jax import lax
from jax.experimental import pallas as pl
from jax.experimental.pallas import tpu as pltpu
from jax.experimental.pallas import tpu_sc as plsc

_L = 16
_W = 128


@functools.lru_cache(maxsize=None)
def _build(cap, push_b, n, d):
    info = plsc.get_sparse_core_info()
    nw = info.num_cores * info.num_subcores
    bpw = n // nw
    waves = bpw // _W
    gpw = _W // _L
    d2 = 2 * d

    mesh = plsc.VectorSubcoreMesh(core_axis_name="c", subcore_axis_name="s")

    @functools.partial(
        pl.kernel,
        mesh=mesh,
        out_type=jax.ShapeDtypeStruct((n * d,), jnp.float32),
        scratch_types=[
            pltpu.VMEM((bpw,), jnp.int32),
            pltpu.VMEM((bpw,), jnp.int32),
            pltpu.VMEM((bpw,), jnp.int32),
            pltpu.VMEM((bpw,), jnp.int32),
            pltpu.VMEM((_L,), jnp.int32),
            pltpu.VMEM((_W, 1, d2), jnp.float32),
            pltpu.VMEM((1, d2), jnp.float32),
            pltpu.VMEM((bpw * d,), jnp.float32),
            pltpu.SemaphoreType.DMA,
        ],
    )
    def sc_kernel(buf_hbm, val_hbm, ptr_hbm, sidx_hbm, out_hbm,
                  idx_v, wm_v, rp_v, hf_v, ptr_v, blk, dmy, os_v, sem):
        wid = lax.axis_index("s") * info.num_cores + lax.axis_index("c")
        base = wid * bpw

        pltpu.sync_copy(sidx_hbm.at[pl.ds(base, bpw)], idx_v)
        pltpu.sync_copy(ptr_hbm, ptr_v)
        ptrv = ptr_v[...]

        zero = jnp.zeros((_L,), jnp.int32)
        one = jnp.ones((_L,), jnp.int32)
        capv = jnp.full((_L,), cap, jnp.int32)
        pbv = jnp.full((_L,), push_b, jnp.int32)
        dv = jnp.full((_L,), d, jnp.int32)

        for t in range(bpw // _L):
            sl = pl.ds(t * _L, _L)
            s = idx_v[sl]
            off = s - ptrv
            off = jnp.where(off < zero, off + capv, off)
            w = off < pbv
            eff = jnp.where(w, off, s)
            wm_v[sl] = jnp.where(w, one, zero)
            rp_v[sl] = eff >> one
            hf_v[sl] = (eff & one) * dv

        for h in range(waves):
            def fire_body(g, carry, h=h):
                j0 = h * _W + g * _L
                rv = rp_v[pl.ds(j0, _L)]
                mv = wm_v[pl.ds(j0, _L)]
                for k in range(_L):
                    r = rv[k]
                    m = mv[k]
                    slot = g * _L + k

                    @pl.when(m == 0)
                    def _(r=r, slot=slot):
                        pltpu.async_copy(
                            buf_hbm.at[pl.ds(r, 1), :], blk.at[slot], sem)

                    @pl.when(m != 0)
                    def _(r=r, slot=slot):
                        pltpu.async_copy(
                            val_hbm.at[pl.ds(r, 1), :], blk.at[slot], sem)

                return carry

            lax.fori_loop(0, gpw, fire_body, 0)

            def drain_body(j, carry):
                pltpu.make_async_copy(buf_hbm.at[pl.ds(0, 1), :], dmy,
                                      sem).wait()
                return carry

            lax.fori_loop(0, _W, drain_body, 0)

            def extract_body(g, carry, h=h):
                j0 = h * _W + g * _L
                hv = hf_v[pl.ds(j0, _L)]
                for k in range(_L):
                    hf = hv[k]
                    slot = g * _L + k
                    for c in range(d // _L):
                        os_v[pl.ds((j0 + k) * d + c * _L, _L)] = (
                            blk[slot, 0, pl.ds(hf + c * _L, _L)])
                return carry

            lax.fori_loop(0, gpw, extract_body, 0)

        pltpu.sync_copy(os_v, out_hbm.at[pl.ds(base * d, bpw * d)])

    return sc_kernel


def kernel(buffer, val, ptr, sample_idx):
    cap, d = buffer.shape
    push_b = val.shape[0]
    n = sample_idx.shape[0]
    ptr_vec = jnp.full((_L,), ptr, dtype=jnp.int32)
    buf2 = buffer.reshape(cap // 2, 2 * d)
    val2 = val.reshape(push_b // 2, 2 * d)
    sc = _build(cap, push_b, n, d)
    out_flat = sc(buf2, val2, ptr_vec, sample_idx.astype(jnp.int32))
    return out_flat.reshape(n, d)


# speedup vs baseline: 2.4347x; 2.4347x over previous
import functools

import jax
import jax.numpy as jnp
from jax import lax
from jax.experimental import pallas as pl
from jax.experimental.pallas import tpu as pltpu
from jax.experimental.pallas import tpu_sc as plsc

_L = 16
_W = 64


@functools.lru_cache(maxsize=None)
def _build(cap, push_b, n, d):
    info = plsc.get_sparse_core_info()
    nw = info.num_cores * info.num_subcores
    bpw = n // nw
    waves = bpw // _W
    gpw = _W // _L

    mesh = plsc.VectorSubcoreMesh(core_axis_name="c", subcore_axis_name="s")

    @functools.partial(
        pl.kernel,
        mesh=mesh,
        out_type=jax.ShapeDtypeStruct((n * d,), jnp.float32),
        compiler_params=pltpu.CompilerParams(use_tc_tiling_on_sc=True),
        scratch_types=[
            pltpu.VMEM((bpw,), jnp.int32),
            pltpu.VMEM((bpw,), jnp.int32),
            pltpu.VMEM((bpw,), jnp.int32),
            pltpu.VMEM((bpw,), jnp.int32),
            pltpu.VMEM((_L,), jnp.int32),
            pltpu.VMEM((_W, 8, d), jnp.float32),
            pltpu.VMEM((8, d), jnp.float32),
            pltpu.VMEM((bpw * d,), jnp.float32),
            pltpu.SemaphoreType.DMA,
        ],
    )
    def sc_kernel(buf_hbm, val_hbm, ptr_hbm, sidx_hbm, out_hbm,
                  idx_v, wm_v, ab_v, rs_v, ptr_v, blk, dmy, os_v, sem):
        wid = lax.axis_index("s") * info.num_cores + lax.axis_index("c")
        base = wid * bpw

        pltpu.sync_copy(sidx_hbm.at[pl.ds(base, bpw)], idx_v)
        pltpu.sync_copy(ptr_hbm, ptr_v)
        ptrv = ptr_v[...]

        zero = jnp.zeros((_L,), jnp.int32)
        one = jnp.ones((_L,), jnp.int32)
        capv = jnp.full((_L,), cap, jnp.int32)
        pbv = jnp.full((_L,), push_b, jnp.int32)
        c3 = jnp.full((_L,), 3, jnp.int32)
        m7 = jnp.full((_L,), 7, jnp.int32)

        for t in range(bpw // _L):
            sl = pl.ds(t * _L, _L)
            s = idx_v[sl]
            off = s - ptrv
            off = jnp.where(off < zero, off + capv, off)
            w = off < pbv
            eff = jnp.where(w, off, s)
            wm_v[sl] = jnp.where(w, one, zero)
            ab_v[sl] = eff >> c3
            rs_v[sl] = eff & m7

        for h in range(waves):
            def fire_body(g, carry, h=h):
                j0 = h * _W + g * _L
                av = ab_v[pl.ds(j0, _L)]
                mv = wm_v[pl.ds(j0, _L)]
                for k in range(_L):
                    a = av[k]
                    m = mv[k]
                    slot = g * _L + k

                    @pl.when(m == 0)
                    def _(a=a, slot=slot):
                        pltpu.async_copy(buf_hbm.at[a], blk.at[slot], sem)

                    @pl.when(m != 0)
                    def _(a=a, slot=slot):
                        pltpu.async_copy(val_hbm.at[a], blk.at[slot], sem)

                return carry

            lax.fori_loop(0, gpw, fire_body, 0)

            def drain_body(j, carry):
                pltpu.make_async_copy(buf_hbm.at[0], dmy, sem).wait()
                return carry

            lax.fori_loop(0, _W, drain_body, 0)

            def extract_body(g, carry, h=h):
                j0 = h * _W + g * _L
                rv = rs_v[pl.ds(j0, _L)]
                for k in range(_L):
                    r = rv[k]
                    slot = g * _L + k
                    for c in range(d // _L):
                        os_v[pl.ds((j0 + k) * d + c * _L, _L)] = (
                            blk[slot, r, pl.ds(c * _L, _L)])
                return carry

            lax.fori_loop(0, gpw, extract_body, 0)

        pltpu.sync_copy(os_v, out_hbm.at[pl.ds(base * d, bpw * d)])

    return sc_kernel


def kernel(buffer, val, ptr, sample_idx):
    cap, d = buffer.shape
    push_b = val.shape[0]
    n = sample_idx.shape[0]
    ptr_vec = jnp.full((_L,), ptr, dtype=jnp.int32)
    buf3 = buffer.reshape(cap // 8, 8, d)
    val3 = val.reshape(push_b // 8, 8, d)
    sc = _build(cap, push_b, n, d)
    out_flat = sc(buf3, val3, ptr_vec, sample_idx.astype(jnp.int32))
    return out_flat.reshape(n, d)
